# baseline (speedup 1.0000x reference)
import functools

import jax
import jax.numpy as jnp
from jax import lax
from jax.experimental import pallas as pl
from jax.experimental.pallas import tpu as pltpu
from jax.experimental.pallas import tpu_sc as plsc

NC = 2
NS = 16
NW = NC * NS
LANES = 16
B = 128


def _pre_body(x_ref, wctxT_ref, bctx_ref, w1_ref, w2_ref, bal_ref,
              ctx_ref, s_ref, t_ref):
    x = x_ref[...]
    ctx_ref[...] = (
        jnp.dot(x, wctxT_ref[...], preferred_element_type=jnp.float32)
        + bctx_ref[...]
    )
    s_ref[...] = (
        jnp.dot(x, w1_ref[...], preferred_element_type=jnp.float32)
        + bal_ref[...]
    )
    t_ref[...] = jnp.dot(x, w2_ref[...], preferred_element_type=jnp.float32)


def _prologue(xp, W_ctx, b_ctx, W_align, b_align, n_pad):
    blk = 512
    grid = n_pad // blk
    wctxT = W_ctx.T
    w1 = W_align[0, :128].reshape(128, 1)
    w2 = W_align[0, 128:].reshape(128, 1)
    bal = b_align.reshape(1, 1)
    bctx = b_ctx.reshape(1, 128)
    return pl.pallas_call(
        _pre_body,
        grid=(grid,),
        in_specs=[
            pl.BlockSpec((blk, 128), lambda i: (i, 0)),
            pl.BlockSpec((128, 128), lambda i: (0, 0)),
            pl.BlockSpec((1, 128), lambda i: (0, 0)),
            pl.BlockSpec((128, 1), lambda i: (0, 0)),
            pl.BlockSpec((128, 1), lambda i: (0, 0)),
            pl.BlockSpec((1, 1), lambda i: (0, 0)),
        ],
        out_specs=[
            pl.BlockSpec((blk, 128), lambda i: (i, 0)),
            pl.BlockSpec((blk, 1), lambda i: (i, 0)),
            pl.BlockSpec((blk, 1), lambda i: (i, 0)),
        ],
        out_shape=[
            jax.ShapeDtypeStruct((n_pad, 128), jnp.float32),
            jax.ShapeDtypeStruct((n_pad, 1), jnp.float32),
            jax.ShapeDtypeStruct((n_pad, 1), jnp.float32),
        ],
    )(xp, wctxT, bctx, w1, w2, bal)


def _make_edge_kernel(E, n_pad, dim):
    npt = n_pad // NW
    wdt = dim + LANES
    mesh = plsc.VectorSubcoreMesh(core_axis_name="c", subcore_axis_name="s")

    @functools.partial(
        pl.kernel,
        out_type=jax.ShapeDtypeStruct((n_pad * wdt,), jnp.float32),
        mesh=mesh,
        scratch_types=[
            pltpu.VMEM((n_pad,), jnp.float32),
            pltpu.VMEM((n_pad,), jnp.float32),
            pltpu.VMEM((B,), jnp.int32),
            pltpu.VMEM((B,), jnp.int32),
            pltpu.VMEM((B,), jnp.int32),
            pltpu.VMEM((B,), jnp.int32),
            pltpu.VMEM((B, dim), jnp.float32),
            pltpu.VMEM((B, dim), jnp.float32),
            pltpu.VMEM((npt * wdt,), jnp.float32),
            pltpu.VMEM((LANES * LANES,), jnp.float32),
            pltpu.VMEM((LANES * LANES,), jnp.int32),
            pltpu.VMEM((48,), jnp.int32),
            pltpu.SemaphoreType.DMA,
            pltpu.SemaphoreType.DMA,
        ],
        compiler_params=pltpu.CompilerParams(needs_layout_passes=False),
    )
    def edge_kernel(dst_hbm, nbr_hbm, s_hbm, t_hbm, ctx_hbm, ebounds_hbm,
                    cpart, s_tab, t_tab, sidx0, gidx0, sidx1, gidx1,
                    rows0, rows1, acc, evb, dlocb, bnds, sem0, sem1):
        cid = lax.axis_index("c")
        sid = lax.axis_index("s")
        wid = cid * NS + sid
        zeros16 = jnp.zeros((LANES,), jnp.float32)
        iota16 = lax.iota(jnp.int32, LANES)
        lane0 = jnp.where(iota16 == 0, jnp.float32(1.0), jnp.float32(0.0))
        base = wid * npt

        pltpu.sync_copy(s_hbm, s_tab)
        pltpu.sync_copy(t_hbm, t_tab)
        pltpu.sync_copy(ebounds_hbm, bnds)

        def _extract(widx):
            v = jnp.int32(0)
            for chunk in range(3):
                bv = bnds[pl.ds(chunk * LANES, LANES)]
                v = v + jnp.sum(jnp.where(chunk * LANES + iota16 == widx,
                                          bv, 0))
            return v
        e_lo = _extract(wid)
        e_hi = _extract(wid + 1)

        def _zrow(j, _):
            acc[pl.ds(j * LANES, LANES)] = zeros16
            return 0
        lax.fori_loop(0, npt * wdt // LANES, _zrow, 0)

        blk_lo = e_lo // B
        blk_hi = (e_hi + B - 1) // B

        def _fetch(blk, sidx, gidx, rows, sem):
            e0 = blk * B
            pltpu.sync_copy(dst_hbm.at[pl.ds(e0, B)], sidx)
            pltpu.sync_copy(nbr_hbm.at[pl.ds(e0, B)], gidx)
            pltpu.async_copy(ctx_hbm.at[gidx], rows, sem)

        def _process(blk, sidx, gidx, rows):
            e0 = blk * B

            def _group(g, _):
                gb = g * LANES
                d16 = sidx[pl.ds(gb, LANES)]
                n16 = gidx[pl.ds(gb, LANES)]
                sv = plsc.load_gather(s_tab, [d16])
                tv = plsc.load_gather(t_tab, [n16])
                a = sv + tv
                a = jnp.where(a >= 0.0, a, a * jnp.float32(0.01))
                eg = e0 + gb + iota16
                valid = (eg >= e_lo) & (eg < e_hi)
                ev = jnp.where(valid, jnp.exp(a), jnp.float32(0.0))
                dloc = jnp.clip(d16 - base, 0, npt - 1)
                bvals = dloc * jnp.int32(wdt)
                for c in range(LANES):
                    caddr = iota16 * LANES + c
                    plsc.store_scatter(evb, [caddr], ev)
                    plsc.store_scatter(dlocb, [caddr], bvals)
                for j in range(LANES):
                    evj = evb[pl.ds(j * LANES, LANES)]
                    addr = dlocb[pl.ds(j * LANES, LANES)] + iota16
                    for k in range(dim // LANES):
                        sl = pl.ds(k * LANES, LANES)
                        plsc.addupdate_scatter(
                            acc, [addr + (k * LANES)], rows[gb + j, sl] * evj)
                    plsc.addupdate_scatter(acc, [addr + dim], lane0 * evj)
                return 0
            lax.fori_loop(0, B // LANES, _group, 0)

        @pl.when(blk_lo < blk_hi)
        def _():
            _fetch(blk_lo, sidx0, gidx0, rows0, sem0)

        bufs = ((sidx0, gidx0, rows0, sem0), (sidx1, gidx1, rows1, sem1))

        def _pair(p, _):
            for half in range(2):
                blk = blk_lo + 2 * p + half
                si, gi_, ro, se = bufs[half]
                nsi, ngi, nro, nse = bufs[1 - half]

                @pl.when(blk < blk_hi)
                def _():
                    @pl.when(blk + 1 < blk_hi)
                    def _():
                        _fetch(blk + 1, nsi, ngi, nro, nse)
                    pltpu.make_async_copy(ctx_hbm.at[gi_], ro, se).wait()
                    _process(blk, si, gi_, ro)
            return 0
        lax.fori_loop(0, (blk_hi - blk_lo + 1) // 2, _pair, 0)

        pltpu.sync_copy(acc, cpart.at[pl.ds(base * wdt, npt * wdt)])

    return edge_kernel


def _post_body(x_ref, cp_ref, wihT_ref, bih_ref,
               whhT_ref, bhh_ref, out_ref):
    x = x_ref[...]
    craw = cp_ref[...][:, 0:128]
    den = cp_ref[...][:, 128:129]
    ct = craw / (den + jnp.float32(1e-16))
    ct = jnp.where(ct > 0.0, ct, jnp.exp(jnp.minimum(ct, 0.0)) - 1.0)
    gi = jnp.dot(ct, wihT_ref[...], preferred_element_type=jnp.float32) + bih_ref[...]
    gh = jnp.dot(x, whhT_ref[...], preferred_element_type=jnp.float32) + bhh_ref[...]
    r = jax.nn.sigmoid(gi[:, 0:128] + gh[:, 0:128])
    z = jax.nn.sigmoid(gi[:, 128:256] + gh[:, 128:256])
    n = jnp.tanh(gi[:, 256:384] + r * gh[:, 256:384])
    h = (1.0 - z) * n + z * x
    out_ref[...] = jnp.maximum(h, 0.0)


def _epilogue(x, cpart, W_ih, b_ih, W_hh, b_hh, N, wdt):
    blk = 400
    grid = N // blk
    return pl.pallas_call(
        _post_body,
        grid=(grid,),
        in_specs=[
            pl.BlockSpec((blk, 128), lambda i: (i, 0)),
            pl.BlockSpec((blk, wdt), lambda i: (i, 0)),
            pl.BlockSpec((128, 384), lambda i: (0, 0)),
            pl.BlockSpec((1, 384), lambda i: (0, 0)),
            pl.BlockSpec((128, 384), lambda i: (0, 0)),
            pl.BlockSpec((1, 384), lambda i: (0, 0)),
        ],
        out_specs=pl.BlockSpec((blk, 128), lambda i: (i, 0)),
        out_shape=jax.ShapeDtypeStruct((N, 128), jnp.float32),
    )(x, cpart, W_ih.T, b_ih.reshape(1, 384), W_hh.T,
      b_hh.reshape(1, 384))


def kernel(node_embeddings, batch_index, neighbour_index, W_align, b_align,
           W_ctx, b_ctx, W_ih, W_hh, b_ih, b_hh):
    N, dim = node_embeddings.shape
    E = batch_index.shape[0]
    n_pad = ((N + 2047) // 2048) * 2048
    assert E % B == 0

    dst = batch_index.astype(jnp.int32)
    nbr = neighbour_index.astype(jnp.int32)
    xp = jnp.pad(node_embeddings, ((0, n_pad - N), (0, 0)))

    ctx, s_col, t_col = _prologue(xp, W_ctx, b_ctx, W_align, b_align, n_pad)
    npt = n_pad // NW
    node_bounds = jnp.arange(NW + 1, dtype=jnp.int32) * npt
    ebounds = jnp.searchsorted(dst, node_bounds).astype(jnp.int32)
    ebounds = jnp.pad(ebounds, (0, 48 - (NW + 1)))
    edge_kernel = _make_edge_kernel(E, n_pad, dim)
    cpart = edge_kernel(dst, nbr, s_col.reshape(n_pad),
                        t_col.reshape(n_pad), ctx, ebounds)
    cpart = cpart.reshape(n_pad, dim + LANES)
    return _epilogue(node_embeddings, cpart, W_ih, b_ih, W_hh, b_hh, N,
                     dim + LANES)


# speedup vs baseline: 1.1361x; 1.1361x over previous
import functools

import jax
import jax.numpy as jnp
from jax import lax
from jax.experimental import pallas as pl
from jax.experimental.pallas import tpu as pltpu
from jax.experimental.pallas import tpu_sc as plsc

NC = 2
NS = 16
NW = NC * NS
LANES = 16
B = 128


def _pre_body(x_ref, wctxT_ref, bctx_ref, w1_ref, w2_ref, bal_ref,
              ctx_ref, s_ref, t_ref):
    x = x_ref[...]
    ctx_ref[...] = (
        jnp.dot(x, wctxT_ref[...], preferred_element_type=jnp.float32)
        + bctx_ref[...]
    )
    s_ref[...] = (
        jnp.dot(x, w1_ref[...], preferred_element_type=jnp.float32)
        + bal_ref[...]
    )
    t_ref[...] = jnp.dot(x, w2_ref[...], preferred_element_type=jnp.float32)


def _prologue(xp, W_ctx, b_ctx, W_align, b_align, n_pad):
    blk = 512
    grid = n_pad // blk
    wctxT = W_ctx.T
    w1 = W_align[0, :128].reshape(128, 1)
    w2 = W_align[0, 128:].reshape(128, 1)
    bal = b_align.reshape(1, 1)
    bctx = b_ctx.reshape(1, 128)
    return pl.pallas_call(
        _pre_body,
        grid=(grid,),
        in_specs=[
            pl.BlockSpec((blk, 128), lambda i: (i, 0)),
            pl.BlockSpec((128, 128), lambda i: (0, 0)),
            pl.BlockSpec((1, 128), lambda i: (0, 0)),
            pl.BlockSpec((128, 1), lambda i: (0, 0)),
            pl.BlockSpec((128, 1), lambda i: (0, 0)),
            pl.BlockSpec((1, 1), lambda i: (0, 0)),
        ],
        out_specs=[
            pl.BlockSpec((blk, 128), lambda i: (i, 0)),
            pl.BlockSpec((blk, 1), lambda i: (i, 0)),
            pl.BlockSpec((blk, 1), lambda i: (i, 0)),
        ],
        out_shape=[
            jax.ShapeDtypeStruct((n_pad, 128), jnp.float32),
            jax.ShapeDtypeStruct((n_pad, 1), jnp.float32),
            jax.ShapeDtypeStruct((n_pad, 1), jnp.float32),
        ],
    )(xp, wctxT, bctx, w1, w2, bal)


def _make_edge_kernel(E, n_pad, dim):
    npt = n_pad // NW
    wdt = dim + LANES
    mesh = plsc.VectorSubcoreMesh(core_axis_name="c", subcore_axis_name="s")

    @functools.partial(
        pl.kernel,
        out_type=jax.ShapeDtypeStruct((n_pad * wdt,), jnp.float32),
        mesh=mesh,
        scratch_types=[
            pltpu.VMEM((n_pad,), jnp.float32),
            pltpu.VMEM((n_pad,), jnp.float32),
            pltpu.VMEM((B,), jnp.int32),
            pltpu.VMEM((B,), jnp.int32),
            pltpu.VMEM((B,), jnp.int32),
            pltpu.VMEM((B,), jnp.int32),
            pltpu.VMEM((B, dim), jnp.float32),
            pltpu.VMEM((B, dim), jnp.float32),
            pltpu.VMEM((npt * wdt,), jnp.float32),
            pltpu.VMEM((LANES,), jnp.float32),
            pltpu.VMEM((LANES,), jnp.int32),
            pltpu.VMEM((48,), jnp.int32),
            pltpu.SemaphoreType.DMA,
            pltpu.SemaphoreType.DMA,
        ],
        compiler_params=pltpu.CompilerParams(needs_layout_passes=False),
    )
    def edge_kernel(dst_hbm, nbr_hbm, s_hbm, t_hbm, ctx_hbm, ebounds_hbm,
                    cpart, s_tab, t_tab, sidx0, gidx0, sidx1, gidx1,
                    rows0, rows1, acc, evb, dlocb, bnds, sem0, sem1):
        cid = lax.axis_index("c")
        sid = lax.axis_index("s")
        wid = cid * NS + sid
        zeros16 = jnp.zeros((LANES,), jnp.float32)
        iota16 = lax.iota(jnp.int32, LANES)
        lane0 = jnp.where(iota16 == 0, jnp.float32(1.0), jnp.float32(0.0))
        base = wid * npt

        pltpu.sync_copy(s_hbm, s_tab)
        pltpu.sync_copy(t_hbm, t_tab)
        pltpu.sync_copy(ebounds_hbm, bnds)

        def _extract(widx):
            v = jnp.int32(0)
            for chunk in range(3):
                bv = bnds[pl.ds(chunk * LANES, LANES)]
                v = v + jnp.sum(jnp.where(chunk * LANES + iota16 == widx,
                                          bv, 0))
            return v
        e_lo = _extract(wid)
        e_hi = _extract(wid + 1)

        def _zrow(j, _):
            acc[pl.ds(j * LANES, LANES)] = zeros16
            return 0
        lax.fori_loop(0, npt * wdt // LANES, _zrow, 0)

        blk_lo = e_lo // B
        blk_hi = (e_hi + B - 1) // B

        def _fetch(blk, sidx, gidx, rows, sem):
            e0 = blk * B
            pltpu.sync_copy(dst_hbm.at[pl.ds(e0, B)], sidx)
            pltpu.sync_copy(nbr_hbm.at[pl.ds(e0, B)], gidx)
            pltpu.async_copy(ctx_hbm.at[gidx], rows, sem)

        def _process(blk, sidx, gidx, rows):
            e0 = blk * B

            def _group(g, _):
                gb = g * LANES
                d16 = sidx[pl.ds(gb, LANES)]
                n16 = gidx[pl.ds(gb, LANES)]
                sv = plsc.load_gather(s_tab, [d16])
                tv = plsc.load_gather(t_tab, [n16])
                a = sv + tv
                a = jnp.where(a >= 0.0, a, a * jnp.float32(0.01))
                eg = e0 + gb + iota16
                valid = (eg >= e_lo) & (eg < e_hi)
                ev = jnp.where(valid, jnp.exp(a), jnp.float32(0.0))
                dloc = jnp.clip(d16 - base, 0, npt - 1)
                for j in range(LANES):
                    onej = iota16 == j
                    evj = jnp.sum(jnp.where(onej, ev, jnp.float32(0.0)))
                    dj = jnp.sum(jnp.where(onej, dloc, 0))
                    addr = dj * jnp.int32(wdt) + iota16
                    for k in range(dim // LANES):
                        sl = pl.ds(k * LANES, LANES)
                        plsc.addupdate_scatter(
                            acc, [addr + (k * LANES)], rows[gb + j, sl] * evj)
                    plsc.addupdate_scatter(acc, [addr + dim], lane0 * evj)
                return 0
            lax.fori_loop(0, B // LANES, _group, 0)

        @pl.when(blk_lo < blk_hi)
        def _():
            _fetch(blk_lo, sidx0, gidx0, rows0, sem0)

        bufs = ((sidx0, gidx0, rows0, sem0), (sidx1, gidx1, rows1, sem1))

        def _pair(p, _):
            for half in range(2):
                blk = blk_lo + 2 * p + half
                si, gi_, ro, se = bufs[half]
                nsi, ngi, nro, nse = bufs[1 - half]

                @pl.when(blk < blk_hi)
                def _():
                    @pl.when(blk + 1 < blk_hi)
                    def _():
                        _fetch(blk + 1, nsi, ngi, nro, nse)
                    pltpu.make_async_copy(ctx_hbm.at[gi_], ro, se).wait()
                    _process(blk, si, gi_, ro)
            return 0
        lax.fori_loop(0, (blk_hi - blk_lo + 1) // 2, _pair, 0)

        pltpu.sync_copy(acc, cpart.at[pl.ds(base * wdt, npt * wdt)])

    return edge_kernel


def _post_body(x_ref, cp_ref, wihT_ref, bih_ref,
               whhT_ref, bhh_ref, out_ref):
    x = x_ref[...]
    craw = cp_ref[...][:, 0:128]
    den = cp_ref[...][:, 128:129]
    ct = craw / (den + jnp.float32(1e-16))
    ct = jnp.where(ct > 0.0, ct, jnp.exp(jnp.minimum(ct, 0.0)) - 1.0)
    gi = jnp.dot(ct, wihT_ref[...], preferred_element_type=jnp.float32) + bih_ref[...]
    gh = jnp.dot(x, whhT_ref[...], preferred_element_type=jnp.float32) + bhh_ref[...]
    r = jax.nn.sigmoid(gi[:, 0:128] + gh[:, 0:128])
    z = jax.nn.sigmoid(gi[:, 128:256] + gh[:, 128:256])
    n = jnp.tanh(gi[:, 256:384] + r * gh[:, 256:384])
    h = (1.0 - z) * n + z * x
    out_ref[...] = jnp.maximum(h, 0.0)


def _epilogue(x, cpart, W_ih, b_ih, W_hh, b_hh, N, wdt):
    blk = 400
    grid = N // blk
    return pl.pallas_call(
        _post_body,
        grid=(grid,),
        in_specs=[
            pl.BlockSpec((blk, 128), lambda i: (i, 0)),
            pl.BlockSpec((blk, wdt), lambda i: (i, 0)),
            pl.BlockSpec((128, 384), lambda i: (0, 0)),
            pl.BlockSpec((1, 384), lambda i: (0, 0)),
            pl.BlockSpec((128, 384), lambda i: (0, 0)),
            pl.BlockSpec((1, 384), lambda i: (0, 0)),
        ],
        out_specs=pl.BlockSpec((blk, 128), lambda i: (i, 0)),
        out_shape=jax.ShapeDtypeStruct((N, 128), jnp.float32),
    )(x, cpart, W_ih.T, b_ih.reshape(1, 384), W_hh.T,
      b_hh.reshape(1, 384))


def kernel(node_embeddings, batch_index, neighbour_index, W_align, b_align,
           W_ctx, b_ctx, W_ih, W_hh, b_ih, b_hh):
    N, dim = node_embeddings.shape
    E = batch_index.shape[0]
    n_pad = ((N + 2047) // 2048) * 2048
    assert E % B == 0

    dst = batch_index.astype(jnp.int32)
    nbr = neighbour_index.astype(jnp.int32)
    xp = jnp.pad(node_embeddings, ((0, n_pad - N), (0, 0)))

    ctx, s_col, t_col = _prologue(xp, W_ctx, b_ctx, W_align, b_align, n_pad)
    npt = n_pad // NW
    node_bounds = jnp.arange(NW + 1, dtype=jnp.int32) * npt
    ebounds = jnp.searchsorted(dst, node_bounds).astype(jnp.int32)
    ebounds = jnp.pad(ebounds, (0, 48 - (NW + 1)))
    edge_kernel = _make_edge_kernel(E, n_pad, dim)
    cpart = edge_kernel(dst, nbr, s_col.reshape(n_pad),
                        t_col.reshape(n_pad), ctx, ebounds)
    cpart = cpart.reshape(n_pad, dim + LANES)
    return _epilogue(node_embeddings, cpart, W_ih, b_ih, W_hh, b_hh, N,
                     dim + LANES)
